# baseline (speedup 1.0000x reference)
import functools

import jax
import jax.numpy as jnp
from jax.experimental import pallas as pl
from jax.experimental.pallas import tpu as pltpu
from jax.experimental.pallas import tpu_sc as plsc

H, S, D = 16, 2048, 64
SINK = 4
NOUT = 32
BQ = 512
BK = 256
NEG = jnp.finfo(jnp.float32).min

NW = 32
R_ALL = H * S
VROW_W = R_ALL // NW
VCHUNK = 128
KSP_LEN = H * NOUT * D + H * SINK * D
VSP_LEN = H * S * NOUT + H * SINK * D
V_TAIL = H * S * NOUT
K_TAIL = H * NOUT * D


def _attn_body(q_ref, k_ref, v_ref, o_ref):
    qi = pl.program_id(1)
    q = (q_ref[0] * 0.125).astype(jnp.bfloat16)
    k = k_ref[0].astype(jnp.bfloat16)
    s = jax.lax.dot_general(q, k, (((1,), (1,)), ((), ())),
                            preferred_element_type=jnp.float32)
    row = qi * BQ + jax.lax.broadcasted_iota(jnp.int32, (BQ, S), 0)
    col = jax.lax.broadcasted_iota(jnp.int32, (BQ, S), 1)
    s = jnp.where(col <= row, s, NEG)
    m = jnp.max(s, axis=1, keepdims=True)
    p = jnp.exp(s - m)
    l = jnp.sum(p, axis=1, keepdims=True)
    o = jax.lax.dot_general(p.astype(jnp.bfloat16),
                            v_ref[0].astype(jnp.bfloat16),
                            (((1,), (0,)), ((), ())),
                            preferred_element_type=jnp.float32)
    o_ref[0] = o / l


def _attention(q, k, v):
    return pl.pallas_call(
        _attn_body,
        grid=(H, S // BQ),
        in_specs=[
            pl.BlockSpec((1, BQ, D), lambda h, i: (h, i, 0)),
            pl.BlockSpec((1, S, D), lambda h, i: (h, 0, 0)),
            pl.BlockSpec((1, S, D), lambda h, i: (h, 0, 0)),
        ],
        out_specs=pl.BlockSpec((1, BQ, D), lambda h, i: (h, i, 0)),
        out_shape=jax.ShapeDtypeStruct((H, S, D), jnp.float32),
    )(q, k, v)


def _topk(sal, n, length, axis):
    shape = (length, 1) if axis == 0 else (1, length)
    iota = jax.lax.broadcasted_iota(jnp.int32, shape, axis)
    slot_row = jax.lax.broadcasted_iota(jnp.int32, (1, n), 1)

    def body(i, carry):
        sal, idx_row = carry
        m = jnp.max(sal)
        pos = jnp.min(jnp.where(sal == m, iota, length))
        idx_row = jnp.where(slot_row == i, pos, idx_row)
        sal = jnp.where(iota == pos, -1.0, sal)
        return sal, idx_row

    return jax.lax.fori_loop(
        0, n, body, (sal, jnp.zeros((1, n), jnp.int32)))


HB1 = 4


def _kv1_body(k_ref, v_ref, kr_ref, vr_ref, keep_ref, drop_ref, ksal, vsal):
    h = pl.program_id(0)
    rowi = jax.lax.broadcasted_iota(jnp.int32, (HB1, S, D), 1)
    khz = jnp.where(rowi < SINK, 0.0, k_ref[...])
    vhz = jnp.where(rowi < SINK, 0.0, v_ref[...])
    ks = jnp.max(jnp.max(jnp.abs(khz), axis=2, keepdims=True), axis=0)
    vs = jnp.max(jnp.max(jnp.abs(vhz), axis=1, keepdims=True), axis=0)

    @pl.when(h == 0)
    def _():
        ksal[...] = ks
        vsal[...] = vs

    @pl.when(h > 0)
    def _():
        ksal[...] = jnp.maximum(ksal[...], ks)
        vsal[...] = jnp.maximum(vsal[...], vs)

    @pl.when(h == H // HB1 - 1)
    def _():
        ksal_f, kr = _topk(ksal[...], NOUT, S, axis=0)
        kr_ref[...] = kr
        keep_ref[...] = (ksal_f >= 0.0).astype(jnp.float32)
        vsal_f, vr = _topk(vsal[...], NOUT, D, axis=1)
        vr_ref[...] = vr
        drop_ref[...] = (vsal_f < 0.0).astype(jnp.float32)


def _kv1(k, v):
    return pl.pallas_call(
        _kv1_body,
        grid=(H // HB1,),
        in_specs=[
            pl.BlockSpec((HB1, S, D), lambda h: (h, 0, 0)),
            pl.BlockSpec((HB1, S, D), lambda h: (h, 0, 0)),
        ],
        out_specs=[
            pl.BlockSpec((1, NOUT), lambda h: (0, 0)),
            pl.BlockSpec((1, NOUT), lambda h: (0, 0)),
            pl.BlockSpec((S, 1), lambda h: (0, 0)),
            pl.BlockSpec((1, D), lambda h: (0, 0)),
        ],
        out_shape=(
            jax.ShapeDtypeStruct((1, NOUT), jnp.int32),
            jax.ShapeDtypeStruct((1, NOUT), jnp.int32),
            jax.ShapeDtypeStruct((S, 1), jnp.float32),
            jax.ShapeDtypeStruct((1, D), jnp.float32),
        ),
        scratch_shapes=[
            pltpu.VMEM((S, 1), jnp.float32),
            pltpu.VMEM((1, D), jnp.float32),
        ],
    )(k, v)


def _kv2_body(k_ref, v_ref, keep_ref, drop_ref,
              kq_ref, kscale_ref, vq_ref, vscale_ref):
    rowi = jax.lax.broadcasted_iota(jnp.int32, (S, D), 0)

    khz = jnp.where(rowi < SINK, 0.0, k_ref[0])
    dense = khz * keep_ref[...]
    absmax = jnp.max(jnp.abs(dense), axis=0, keepdims=True)
    scale = jnp.maximum(absmax, 1e-8) / 127.0
    kq_ref[0] = jnp.clip(jnp.round(dense / scale), -127.0, 127.0
                         ).astype(jnp.int8)
    kscale_ref[0] = scale

    vhz = jnp.where(rowi < SINK, 0.0, v_ref[0])
    dense_v = vhz * (1.0 - drop_ref[...])
    vabs = jnp.max(jnp.abs(dense_v), axis=1, keepdims=True)
    vsc = jnp.maximum(vabs, 1e-8) / 127.0
    vq_ref[0] = jnp.clip(jnp.round(dense_v / vsc), -127.0, 127.0
                         ).astype(jnp.int8)
    vscale_ref[0] = vsc


def _kv2(k, v, keep, drop):
    return pl.pallas_call(
        _kv2_body,
        grid=(H,),
        in_specs=[
            pl.BlockSpec((1, S, D), lambda h: (h, 0, 0)),
            pl.BlockSpec((1, S, D), lambda h: (h, 0, 0)),
            pl.BlockSpec((S, 1), lambda h: (0, 0)),
            pl.BlockSpec((1, D), lambda h: (0, 0)),
        ],
        out_specs=[
            pl.BlockSpec((1, S, D), lambda h: (h, 0, 0)),
            pl.BlockSpec((1, 1, D), lambda h: (h, 0, 0)),
            pl.BlockSpec((1, S, D), lambda h: (h, 0, 0)),
            pl.BlockSpec((1, S, 1), lambda h: (h, 0, 0)),
        ],
        out_shape=(
            jax.ShapeDtypeStruct((H, S, D), jnp.int8),
            jax.ShapeDtypeStruct((H, 1, D), jnp.float32),
            jax.ShapeDtypeStruct((H, S, D), jnp.int8),
            jax.ShapeDtypeStruct((H, S, 1), jnp.float32),
        ),
    )(k, v, keep, drop)


VSINK_CHUNKS = (H * SINK) // 16


def _iota16():
    return jax.lax.broadcasted_iota(jnp.int32, (16,), 0)


def _dg(vec, idx):
    return jax.lax.gather(
        vec, idx.reshape(16, 1),
        jax.lax.GatherDimensionNumbers(offset_dims=(),
                                       collapsed_slice_dims=(0,),
                                       start_index_map=(0,)),
        (1,), mode=jax.lax.GatherScatterMode.PROMISE_IN_BOUNDS)


def _splat(vec, j):
    return _dg(vec, jnp.full((16,), j, jnp.int32))


def _emit_k_isolate(kw_hbm, rowidx, off, idxv, rows_w, buf_f, buf_i,
                    val_out, idx_out, sem):
    lane = _iota16()
    idxv[...] = rowidx >> 1
    pltpu.async_copy(kw_hbm.at[idxv], rows_w, sem).wait()
    for j in range(16):
        base = _splat(rowidx, j)
        parf = (base & 1).astype(jnp.float32)
        keepf = jnp.minimum(jnp.maximum((base & (S - 1)) - (SINK - 1), 0),
                            1).astype(jnp.float32)
        baseD = base * D
        for q in range(4):
            left = rows_w[j, pl.ds(q * 16, 16)]
            right = rows_w[j, pl.ds(D + q * 16, 16)]
            seg = (left * (1.0 - parf) + right * parf) * keepf
            buf_f[pl.ds(j * D + q * 16, 16)] = seg
            buf_i[pl.ds(j * D + q * 16, 16)] = baseD + (q * 16 + lane)
    pltpu.sync_copy(buf_f, val_out.at[pl.ds(off, 16 * D)])
    pltpu.sync_copy(buf_i, idx_out.at[pl.ds(off, 16 * D)])


def _emit_rowchunk(src_hbm, rowidx, rjs, zero_sink, off,
                   val_out, idx_out, rows_v, buf_f, buf_i, sem):
    lane = _iota16()
    copies = []
    for j in range(16):
        copies.append(pltpu.async_copy(src_hbm.at[pl.ds(rjs[j], 1)],
                                       rows_v.at[pl.ds(j, 1)], sem))
    for cp in copies:
        cp.wait()
    for j in range(16):
        base = _splat(rowidx, j) * D
        for q in range(4):
            seg = rows_v[j, pl.ds(q * 16, 16)]
            buf_f[pl.ds(j * D + q * 16, 16)] = seg
            buf_i[pl.ds(j * D + q * 16, 16)] = base + (q * 16 + lane)
    pltpu.sync_copy(buf_f, val_out.at[pl.ds(off, 16 * D)])
    pltpu.sync_copy(buf_i, idx_out.at[pl.ds(off, 16 * D)])


def _sc_sideband(k2, v2, kidx, vidx, tab):
    n_cores = 2
    mesh = plsc.VectorSubcoreMesh(core_axis_name="c", subcore_axis_name="s",
                                  num_cores=n_cores)

    @functools.partial(
        pl.kernel,
        out_type=(
            jax.ShapeDtypeStruct((KSP_LEN,), jnp.float32),
            jax.ShapeDtypeStruct((KSP_LEN,), jnp.int32),
            jax.ShapeDtypeStruct((VSP_LEN,), jnp.float32),
            jax.ShapeDtypeStruct((VSP_LEN,), jnp.int32),
        ),
        mesh=mesh,
        scratch_types=[
            pltpu.VMEM((NOUT,), jnp.int32),
            pltpu.VMEM((NOUT,), jnp.int32),
            pltpu.VMEM((16,), jnp.int32),
            pltpu.VMEM((16,), jnp.int32),
            pltpu.VMEM((16,), jnp.int32),
            pltpu.VMEM((16, 2 * D), jnp.float32),
            pltpu.VMEM((16, D), jnp.float32),
            pltpu.VMEM((16 * D,), jnp.float32),
            pltpu.VMEM((16 * D,), jnp.int32),
            pltpu.VMEM((VCHUNK, D), jnp.float32),
            pltpu.VMEM((VCHUNK * NOUT,), jnp.float32),
            pltpu.VMEM((VCHUNK * NOUT,), jnp.int32),
            pltpu.SemaphoreType.DMA,
        ],
    )
    def sc(k_hbm, kw_hbm, v_hbm, kidx_hbm, vidx_hbm, tab_hbm,
           kspv_out, kspi_out, vspv_out, vspi_out,
           kidx_v, vidx_v, wit_v, rit_v, idxv, rows_w, rows_v, buf_f, buf_i,
           vbuf, obuf, oidx, sem):
        w = jax.lax.axis_index("s") * n_cores + jax.lax.axis_index("c")
        pltpu.sync_copy(kidx_hbm, kidx_v)
        pltpu.sync_copy(vidx_hbm, vidx_v)
        pltpu.sync_copy(tab_hbm.at[pl.ds(w * 16, 16)], wit_v)
        it16 = _iota16()
        r = wit_v[...]

        k0 = kidx_v[pl.ds(0, 16)]
        k1 = kidx_v[pl.ds(16, 16)]
        slot = r & (NOUT - 1)
        tok = jnp.where(slot < 16, _dg(k0, slot & 15), _dg(k1, slot & 15))
        rowidx = (r >> 5) * S + tok
        _emit_k_isolate(kw_hbm, rowidx, w * (16 * D), idxv, rows_w,
                        buf_f, buf_i, kspv_out, kspi_out, sem)

        @pl.when(w < VSINK_CHUNKS)
        def _():
            srow = (r >> 2) * S + (r & (SINK - 1))
            sjs = [((w * 16 + j) // SINK) * S + (w * 16 + j) % SINK
                   for j in range(16)]
            _emit_rowchunk(k_hbm, srow, sjs, False, K_TAIL + w * (16 * D),
                           kspv_out, kspi_out, rows_v, buf_f, buf_i, sem)
            _emit_rowchunk(v_hbm, srow, sjs, False, V_TAIL + w * (16 * D),
                           vspv_out, vspi_out, rows_v, buf_f, buf_i, sem)

        def vchunk(c, _):
            c0 = vidx_v[pl.ds(0, 16)]
            c1 = vidx_v[pl.ds(16, 16)]
            c0l, c1l = c0 & 15, c1 & 15
            zero = jnp.zeros((16,), jnp.int32)
            one = jnp.ones((16,), jnp.int32)
            m0 = [jnp.maximum(one - jnp.abs((c0 >> 4) - q), zero
                              ).astype(jnp.float32) for q in range(4)]
            m1 = [jnp.maximum(one - jnp.abs((c1 >> 4) - q), zero
                              ).astype(jnp.float32) for q in range(4)]
            row0 = w * VROW_W + c * VCHUNK
            pltpu.sync_copy(v_hbm.at[pl.ds(row0, VCHUNK)], vbuf)
            pltpu.sync_copy(tab_hbm.at[pl.ds(row0, 16)], rit_v)
            base = rit_v[...] - _iota16()

            for i in range(VCHUNK):
                rv = [vbuf[i, pl.ds(q * 16, 16)] for q in range(4)]
                a = (_dg(rv[0], c0l) * m0[0] + _dg(rv[1], c0l) * m0[1]
                     + _dg(rv[2], c0l) * m0[2] + _dg(rv[3], c0l) * m0[3])
                b = (_dg(rv[0], c1l) * m1[0] + _dg(rv[1], c1l) * m1[1]
                     + _dg(rv[2], c1l) * m1[2] + _dg(rv[3], c1l) * m1[3])
                rgv = base + i
                keepf = jnp.minimum(jnp.maximum(
                    (rgv & (S - 1)) - (SINK - 1), 0), 1).astype(jnp.float32)
                a = a * keepf
                b = b * keepf
                obuf[pl.ds(i * NOUT, 16)] = a
                obuf[pl.ds(i * NOUT + 16, 16)] = b
                oidx[pl.ds(i * NOUT, 16)] = rgv * D + c0
                oidx[pl.ds(i * NOUT + 16, 16)] = rgv * D + c1

            pltpu.sync_copy(obuf, vspv_out.at[pl.ds(row0 * NOUT,
                                                    VCHUNK * NOUT)])
            pltpu.sync_copy(oidx, vspi_out.at[pl.ds(row0 * NOUT,
                                                    VCHUNK * NOUT)])
            return 0

        jax.lax.fori_loop(0, VROW_W // VCHUNK, vchunk, 0)

    return sc(k2, k2.reshape(R_ALL // 2, 2 * D), v2, kidx, vidx, tab)


def kernel(q_tensor, k_tensor, v_tensor):
    q = q_tensor.reshape(H, S, D)
    k = k_tensor.reshape(H, S, D)
    v = v_tensor.reshape(H, S, D)

    attn = _attention(q, k, v).reshape(1, H, S, D)
    kr, vr, keep, drop = _kv1(k, v)
    kq, kscale, vq, vscale = _kv2(k, v, keep, drop)
    tab = jnp.arange(R_ALL, dtype=jnp.int32)
    k_sp_val, k_sp_idx, v_sp_val, v_sp_idx = _sc_sideband(
        k.reshape(R_ALL, D), v.reshape(R_ALL, D),
        kr.reshape(NOUT), vr.reshape(NOUT), tab)

    return (attn,
            kq.reshape(1, H, S, D),
            kscale.reshape(1, H, 1, D),
            k_sp_val, k_sp_idx,
            vq.reshape(1, H, S, D),
            vscale.reshape(1, H, S, 1),
            v_sp_val, v_sp_idx)


# speedup vs baseline: 1.0927x; 1.0927x over previous
import functools

import jax
import jax.numpy as jnp
from jax.experimental import pallas as pl
from jax.experimental.pallas import tpu as pltpu
from jax.experimental.pallas import tpu_sc as plsc

H, S, D = 16, 2048, 64
SINK = 4
NOUT = 32
BQ = 256
BK = 256
NEG = jnp.finfo(jnp.float32).min

NW = 32
R_ALL = H * S
VROW_W = R_ALL // NW
VCHUNK = 128
KSP_LEN = H * NOUT * D + H * SINK * D
VSP_LEN = H * S * NOUT + H * SINK * D
V_TAIL = H * S * NOUT
K_TAIL = H * NOUT * D


def _attn_body(q_ref, k_ref, v_ref, o_ref):
    qi = pl.program_id(1)
    q = (q_ref[0] * 0.125).astype(jnp.bfloat16)
    k = k_ref[0].astype(jnp.bfloat16)
    s = jax.lax.dot_general(q, k, (((1,), (1,)), ((), ())),
                            preferred_element_type=jnp.float32)
    row = qi * BQ + jax.lax.broadcasted_iota(jnp.int32, (BQ, S), 0)
    col = jax.lax.broadcasted_iota(jnp.int32, (BQ, S), 1)
    s = jnp.where(col <= row, s, NEG)
    m = jnp.max(s, axis=1, keepdims=True)
    p = jnp.exp(s - m)
    l = jnp.sum(p, axis=1, keepdims=True)
    o = jax.lax.dot_general(p.astype(jnp.bfloat16),
                            v_ref[0].astype(jnp.bfloat16),
                            (((1,), (0,)), ((), ())),
                            preferred_element_type=jnp.float32)
    o_ref[0] = o / l


def _attention(q, k, v):
    return pl.pallas_call(
        _attn_body,
        grid=(H, S // BQ),
        in_specs=[
            pl.BlockSpec((1, BQ, D), lambda h, i: (h, i, 0)),
            pl.BlockSpec((1, S, D), lambda h, i: (h, 0, 0)),
            pl.BlockSpec((1, S, D), lambda h, i: (h, 0, 0)),
        ],
        out_specs=pl.BlockSpec((1, BQ, D), lambda h, i: (h, i, 0)),
        out_shape=jax.ShapeDtypeStruct((H, S, D), jnp.float32),
    )(q, k, v)


def _topk(sal, n, length, axis):
    shape = (length, 1) if axis == 0 else (1, length)
    iota = jax.lax.broadcasted_iota(jnp.int32, shape, axis)
    slot_row = jax.lax.broadcasted_iota(jnp.int32, (1, n), 1)

    def body(i, carry):
        sal, idx_row = carry
        m = jnp.max(sal)
        pos = jnp.min(jnp.where(sal == m, iota, length))
        idx_row = jnp.where(slot_row == i, pos, idx_row)
        sal = jnp.where(iota == pos, -1.0, sal)
        return sal, idx_row

    return jax.lax.fori_loop(
        0, n, body, (sal, jnp.zeros((1, n), jnp.int32)))


HB1 = 4


def _kv1_body(k_ref, v_ref, kr_ref, vr_ref, keep_ref, drop_ref, ksal, vsal):
    h = pl.program_id(0)
    rowi = jax.lax.broadcasted_iota(jnp.int32, (HB1, S, D), 1)
    khz = jnp.where(rowi < SINK, 0.0, k_ref[...])
    vhz = jnp.where(rowi < SINK, 0.0, v_ref[...])
    ks = jnp.max(jnp.max(jnp.abs(khz), axis=2, keepdims=True), axis=0)
    vs = jnp.max(jnp.max(jnp.abs(vhz), axis=1, keepdims=True), axis=0)

    @pl.when(h == 0)
    def _():
        ksal[...] = ks
        vsal[...] = vs

    @pl.when(h > 0)
    def _():
        ksal[...] = jnp.maximum(ksal[...], ks)
        vsal[...] = jnp.maximum(vsal[...], vs)

    @pl.when(h == H // HB1 - 1)
    def _():
        ksal_f, kr = _topk(ksal[...], NOUT, S, axis=0)
        kr_ref[...] = kr
        keep_ref[...] = (ksal_f >= 0.0).astype(jnp.float32)
        vsal_f, vr = _topk(vsal[...], NOUT, D, axis=1)
        vr_ref[...] = vr
        drop_ref[...] = (vsal_f < 0.0).astype(jnp.float32)


def _kv1(k, v):
    return pl.pallas_call(
        _kv1_body,
        grid=(H // HB1,),
        in_specs=[
            pl.BlockSpec((HB1, S, D), lambda h: (h, 0, 0)),
            pl.BlockSpec((HB1, S, D), lambda h: (h, 0, 0)),
        ],
        out_specs=[
            pl.BlockSpec((1, NOUT), lambda h: (0, 0)),
            pl.BlockSpec((1, NOUT), lambda h: (0, 0)),
            pl.BlockSpec((S, 1), lambda h: (0, 0)),
            pl.BlockSpec((1, D), lambda h: (0, 0)),
        ],
        out_shape=(
            jax.ShapeDtypeStruct((1, NOUT), jnp.int32),
            jax.ShapeDtypeStruct((1, NOUT), jnp.int32),
            jax.ShapeDtypeStruct((S, 1), jnp.float32),
            jax.ShapeDtypeStruct((1, D), jnp.float32),
        ),
        scratch_shapes=[
            pltpu.VMEM((S, 1), jnp.float32),
            pltpu.VMEM((1, D), jnp.float32),
        ],
    )(k, v)


def _kv2_body(k_ref, v_ref, keep_ref, drop_ref,
              kq_ref, kscale_ref, vq_ref, vscale_ref):
    rowi = jax.lax.broadcasted_iota(jnp.int32, (S, D), 0)

    khz = jnp.where(rowi < SINK, 0.0, k_ref[0])
    dense = khz * keep_ref[...]
    absmax = jnp.max(jnp.abs(dense), axis=0, keepdims=True)
    scale = jnp.maximum(absmax, 1e-8) / 127.0
    kq_ref[0] = jnp.clip(jnp.round(dense / scale), -127.0, 127.0
                         ).astype(jnp.int8)
    kscale_ref[0] = scale

    vhz = jnp.where(rowi < SINK, 0.0, v_ref[0])
    dense_v = vhz * (1.0 - drop_ref[...])
    vabs = jnp.max(jnp.abs(dense_v), axis=1, keepdims=True)
    vsc = jnp.maximum(vabs, 1e-8) / 127.0
    vq_ref[0] = jnp.clip(jnp.round(dense_v / vsc), -127.0, 127.0
                         ).astype(jnp.int8)
    vscale_ref[0] = vsc


def _kv2(k, v, keep, drop):
    return pl.pallas_call(
        _kv2_body,
        grid=(H,),
        in_specs=[
            pl.BlockSpec((1, S, D), lambda h: (h, 0, 0)),
            pl.BlockSpec((1, S, D), lambda h: (h, 0, 0)),
            pl.BlockSpec((S, 1), lambda h: (0, 0)),
            pl.BlockSpec((1, D), lambda h: (0, 0)),
        ],
        out_specs=[
            pl.BlockSpec((1, S, D), lambda h: (h, 0, 0)),
            pl.BlockSpec((1, 1, D), lambda h: (h, 0, 0)),
            pl.BlockSpec((1, S, D), lambda h: (h, 0, 0)),
            pl.BlockSpec((1, S, 1), lambda h: (h, 0, 0)),
        ],
        out_shape=(
            jax.ShapeDtypeStruct((H, S, D), jnp.int8),
            jax.ShapeDtypeStruct((H, 1, D), jnp.float32),
            jax.ShapeDtypeStruct((H, S, D), jnp.int8),
            jax.ShapeDtypeStruct((H, S, 1), jnp.float32),
        ),
    )(k, v, keep, drop)


VSINK_CHUNKS = (H * SINK) // 16


def _iota16():
    return jax.lax.broadcasted_iota(jnp.int32, (16,), 0)


def _dg(vec, idx):
    return jax.lax.gather(
        vec, idx.reshape(16, 1),
        jax.lax.GatherDimensionNumbers(offset_dims=(),
                                       collapsed_slice_dims=(0,),
                                       start_index_map=(0,)),
        (1,), mode=jax.lax.GatherScatterMode.PROMISE_IN_BOUNDS)


def _splat(vec, j):
    return _dg(vec, jnp.full((16,), j, jnp.int32))


def _emit_k_isolate(kw_hbm, rowidx, off, idxv, rows_w, buf_f, buf_i,
                    val_out, idx_out, sem):
    lane = _iota16()
    idxv[...] = rowidx >> 1
    pltpu.async_copy(kw_hbm.at[idxv], rows_w, sem).wait()
    for j in range(16):
        base = _splat(rowidx, j)
        parf = (base & 1).astype(jnp.float32)
        keepf = jnp.minimum(jnp.maximum((base & (S - 1)) - (SINK - 1), 0),
                            1).astype(jnp.float32)
        baseD = base * D
        for q in range(4):
            left = rows_w[j, pl.ds(q * 16, 16)]
            right = rows_w[j, pl.ds(D + q * 16, 16)]
            seg = (left * (1.0 - parf) + right * parf) * keepf
            buf_f[pl.ds(j * D + q * 16, 16)] = seg
            buf_i[pl.ds(j * D + q * 16, 16)] = baseD + (q * 16 + lane)
    pltpu.sync_copy(buf_f, val_out.at[pl.ds(off, 16 * D)])
    pltpu.sync_copy(buf_i, idx_out.at[pl.ds(off, 16 * D)])


def _emit_rowchunk(src_hbm, rowidx, rjs, zero_sink, off,
                   val_out, idx_out, rows_v, buf_f, buf_i, sem):
    lane = _iota16()
    copies = []
    for j in range(16):
        copies.append(pltpu.async_copy(src_hbm.at[pl.ds(rjs[j], 1)],
                                       rows_v.at[pl.ds(j, 1)], sem))
    for cp in copies:
        cp.wait()
    for j in range(16):
        base = _splat(rowidx, j) * D
        for q in range(4):
            seg = rows_v[j, pl.ds(q * 16, 16)]
            buf_f[pl.ds(j * D + q * 16, 16)] = seg
            buf_i[pl.ds(j * D + q * 16, 16)] = base + (q * 16 + lane)
    pltpu.sync_copy(buf_f, val_out.at[pl.ds(off, 16 * D)])
    pltpu.sync_copy(buf_i, idx_out.at[pl.ds(off, 16 * D)])


def _sc_sideband(k2, v2, kidx, vidx, tab):
    n_cores = 2
    mesh = plsc.VectorSubcoreMesh(core_axis_name="c", subcore_axis_name="s",
                                  num_cores=n_cores)

    @functools.partial(
        pl.kernel,
        out_type=(
            jax.ShapeDtypeStruct((KSP_LEN,), jnp.float32),
            jax.ShapeDtypeStruct((KSP_LEN,), jnp.int32),
            jax.ShapeDtypeStruct((VSP_LEN,), jnp.float32),
            jax.ShapeDtypeStruct((VSP_LEN,), jnp.int32),
        ),
        mesh=mesh,
        scratch_types=[
            pltpu.VMEM((NOUT,), jnp.int32),
            pltpu.VMEM((NOUT,), jnp.int32),
            pltpu.VMEM((16,), jnp.int32),
            pltpu.VMEM((16,), jnp.int32),
            pltpu.VMEM((16,), jnp.int32),
            pltpu.VMEM((16, 2 * D), jnp.float32),
            pltpu.VMEM((16, D), jnp.float32),
            pltpu.VMEM((16 * D,), jnp.float32),
            pltpu.VMEM((16 * D,), jnp.int32),
            pltpu.VMEM((VCHUNK, D), jnp.float32),
            pltpu.VMEM((VCHUNK * NOUT,), jnp.float32),
            pltpu.VMEM((VCHUNK * NOUT,), jnp.int32),
            pltpu.SemaphoreType.DMA,
        ],
    )
    def sc(k_hbm, kw_hbm, v_hbm, kidx_hbm, vidx_hbm, tab_hbm,
           kspv_out, kspi_out, vspv_out, vspi_out,
           kidx_v, vidx_v, wit_v, rit_v, idxv, rows_w, rows_v, buf_f, buf_i,
           vbuf, obuf, oidx, sem):
        w = jax.lax.axis_index("s") * n_cores + jax.lax.axis_index("c")
        pltpu.sync_copy(kidx_hbm, kidx_v)
        pltpu.sync_copy(vidx_hbm, vidx_v)
        pltpu.sync_copy(tab_hbm.at[pl.ds(w * 16, 16)], wit_v)
        it16 = _iota16()
        r = wit_v[...]

        k0 = kidx_v[pl.ds(0, 16)]
        k1 = kidx_v[pl.ds(16, 16)]
        slot = r & (NOUT - 1)
        tok = jnp.where(slot < 16, _dg(k0, slot & 15), _dg(k1, slot & 15))
        rowidx = (r >> 5) * S + tok
        _emit_k_isolate(kw_hbm, rowidx, w * (16 * D), idxv, rows_w,
                        buf_f, buf_i, kspv_out, kspi_out, sem)

        @pl.when(w < VSINK_CHUNKS)
        def _():
            srow = (r >> 2) * S + (r & (SINK - 1))
            sjs = [((w * 16 + j) // SINK) * S + (w * 16 + j) % SINK
                   for j in range(16)]
            _emit_rowchunk(k_hbm, srow, sjs, False, K_TAIL + w * (16 * D),
                           kspv_out, kspi_out, rows_v, buf_f, buf_i, sem)
            _emit_rowchunk(v_hbm, srow, sjs, False, V_TAIL + w * (16 * D),
                           vspv_out, vspi_out, rows_v, buf_f, buf_i, sem)

        def vchunk(c, _):
            c0 = vidx_v[pl.ds(0, 16)]
            c1 = vidx_v[pl.ds(16, 16)]
            c0l, c1l = c0 & 15, c1 & 15
            zero = jnp.zeros((16,), jnp.int32)
            one = jnp.ones((16,), jnp.int32)
            m0 = [jnp.maximum(one - jnp.abs((c0 >> 4) - q), zero
                              ).astype(jnp.float32) for q in range(4)]
            m1 = [jnp.maximum(one - jnp.abs((c1 >> 4) - q), zero
                              ).astype(jnp.float32) for q in range(4)]
            row0 = w * VROW_W + c * VCHUNK
            pltpu.sync_copy(v_hbm.at[pl.ds(row0, VCHUNK)], vbuf)
            pltpu.sync_copy(tab_hbm.at[pl.ds(row0, 16)], rit_v)
            base = rit_v[...] - _iota16()

            for i in range(VCHUNK):
                rv = [vbuf[i, pl.ds(q * 16, 16)] for q in range(4)]
                a = (_dg(rv[0], c0l) * m0[0] + _dg(rv[1], c0l) * m0[1]
                     + _dg(rv[2], c0l) * m0[2] + _dg(rv[3], c0l) * m0[3])
                b = (_dg(rv[0], c1l) * m1[0] + _dg(rv[1], c1l) * m1[1]
                     + _dg(rv[2], c1l) * m1[2] + _dg(rv[3], c1l) * m1[3])
                rgv = base + i
                keepf = jnp.minimum(jnp.maximum(
                    (rgv & (S - 1)) - (SINK - 1), 0), 1).astype(jnp.float32)
                a = a * keepf
                b = b * keepf
                obuf[pl.ds(i * NOUT, 16)] = a
                obuf[pl.ds(i * NOUT + 16, 16)] = b
                oidx[pl.ds(i * NOUT, 16)] = rgv * D + c0
                oidx[pl.ds(i * NOUT + 16, 16)] = rgv * D + c1

            pltpu.sync_copy(obuf, vspv_out.at[pl.ds(row0 * NOUT,
                                                    VCHUNK * NOUT)])
            pltpu.sync_copy(oidx, vspi_out.at[pl.ds(row0 * NOUT,
                                                    VCHUNK * NOUT)])
            return 0

        jax.lax.fori_loop(0, VROW_W // VCHUNK, vchunk, 0)

    return sc(k2, k2.reshape(R_ALL // 2, 2 * D), v2, kidx, vidx, tab)


def kernel(q_tensor, k_tensor, v_tensor):
    q = q_tensor.reshape(H, S, D)
    k = k_tensor.reshape(H, S, D)
    v = v_tensor.reshape(H, S, D)

    attn = _attention(q, k, v).reshape(1, H, S, D)
    kr, vr, keep, drop = _kv1(k, v)
    kq, kscale, vq, vscale = _kv2(k, v, keep, drop)
    tab = jnp.arange(R_ALL, dtype=jnp.int32)
    k_sp_val, k_sp_idx, v_sp_val, v_sp_idx = _sc_sideband(
        k.reshape(R_ALL, D), v.reshape(R_ALL, D),
        kr.reshape(NOUT), vr.reshape(NOUT), tab)

    return (attn,
            kq.reshape(1, H, S, D),
            kscale.reshape(1, H, 1, D),
            k_sp_val, k_sp_idx,
            vq.reshape(1, H, S, D),
            vscale.reshape(1, H, S, 1),
            v_sp_val, v_sp_idx)
